# baseline (speedup 1.0000x reference)
import functools
import numpy as np
import jax
import jax.numpy as jnp
from jax import lax
from jax.experimental import pallas as pl
from jax.experimental.pallas import tpu as pltpu
from jax.experimental.pallas import tpu_sc as plsc

N_LEVELS = 16
F = 2
BASE = 16
TARGET = 2048
LOG2_T = 19
T = 2 ** LOG2_T
DIM = 3
N_PTS = 131072
GRID_OUT = 64
HIDDEN = (GRID_OUT + 15) // 16 * 16
SCALE = np.exp2(np.log2(TARGET / BASE) / (N_LEVELS - 1))
RES = [int(np.floor(BASE * SCALE ** l)) for l in range(N_LEVELS)]
DENSE = [(r + 1) ** 3 <= T for r in RES]
P1, P2 = 2654435761, 805459861
ENC = N_LEVELS * F

NC, NS, LANES = 2, 16, 16
NW = NC * NS
PW = N_PTS // NW
CH = 16
NCHUNK = PW // CH
HBUF = 512
FLUSH_EVERY = HBUF // CH

TBL_ELEMS = N_LEVELS * T * F
RL_CHF = 16384
RL_SPAN = TBL_ELEMS // NW
RL_NCH = RL_SPAN // RL_CHF


def _relayout_body(tin_hbm, tout_hbm, bin_v, bout_v, sem):
    wid = lax.axis_index("s") * NC + lax.axis_index("c")
    base = wid * RL_SPAN
    iota = lax.iota(jnp.int32, LANES)

    def chunk(k, carry):
        off = base + k * RL_CHF
        pltpu.sync_copy(tin_hbm.at[pl.ds(off, RL_CHF)], bin_v)

        def block(b, carry2):
            b256 = b * 256

            def eight(m, carry3):
                m16 = b256 + m * 16
                f0 = bin_v[pl.ds(m16, LANES)]
                f1 = bin_v[pl.ds(m16 + 128, LANES)]
                pos = b256 + m * 32 + 2 * iota
                plsc.store_scatter(bout_v, [pos], f0)
                plsc.store_scatter(bout_v, [pos + 1], f1)
                return carry3

            return lax.fori_loop(0, 8, eight, carry2)

        lax.fori_loop(0, RL_CHF // 256, block, 0)
        pltpu.sync_copy(bout_v, tout_hbm.at[pl.ds(off, RL_CHF)])
        return carry

    lax.fori_loop(0, RL_NCH, chunk, 0)


@jax.jit
def _relayout(tflat):
    mesh = plsc.VectorSubcoreMesh(core_axis_name="c", subcore_axis_name="s",
                                  num_cores=NC, num_subcores=NS)
    return pl.kernel(
        _relayout_body,
        out_type=jax.ShapeDtypeStruct((TBL_ELEMS,), jnp.float32),
        mesh=mesh,
        compiler_params=pltpu.CompilerParams(needs_layout_passes=False,
                                             use_tc_tiling_on_sc=False),
        scratch_types=[
            pltpu.VMEM((RL_CHF,), jnp.float32),
            pltpu.VMEM((RL_CHF,), jnp.float32),
            pltpu.SemaphoreType.DMA,
        ],
    )(tflat)


N_STAGED = 2
STAGED_ROWS = [(RES[l] + 1) ** 3 * F // 8 + 8 for l in range(N_STAGED)]


def _make_encode_body(x_off, pw):
    nchunk = pw // CH

    def _encode_body(xt_hbm, table_hbm, h_hbm, xv, wb, colb, hbuf, sem,
                     *lvl_scratch):
        idxrefs = lvl_scratch[:N_LEVELS]
        rowrefs = lvl_scratch[N_LEVELS:2 * N_LEVELS - N_STAGED]
        strefs = lvl_scratch[2 * N_LEVELS - N_STAGED:]
        wid = lax.axis_index("s") * NC + lax.axis_index("c")
        base = wid * pw
        pltpu.sync_copy(xt_hbm.at[:, pl.ds(x_off + base, pw)], xv)
        for l in range(N_STAGED):
            pltpu.sync_copy(
                table_hbm.at[pl.ds(l * (T * F // 8), STAGED_ROWS[l])],
                strefs[l])
        iota = lax.iota(jnp.int32, LANES)

        def chunk_body(k, carry):
            off = k * CH
            xc = xv[0, pl.ds(off, CH)]
            yc = xv[1, pl.ds(off, CH)]
            zc = xv[2, pl.ds(off, CH)]

            for l in range(N_LEVELS):
                res = RES[l]
                resf = float(res)
                px = xc * resf
                py = yc * resf
                pz = zc * resf
                ix = px.astype(jnp.int32)
                iy = py.astype(jnp.int32)
                iz = pz.astype(jnp.int32)
                fx = px - ix.astype(jnp.float32)
                fy = py - iy.astype(jnp.float32)
                fz = pz - iz.astype(jnp.float32)
                x0, x1 = ix, jnp.minimum(ix + 1, res)
                y0, y1 = iy, jnp.minimum(iy + 1, res)
                z0, z1 = iz, jnp.minimum(iz + 1, res)
                wx0, wx1 = 1.0 - fx, fx
                wy0, wy1 = 1.0 - fy, fy
                wz0, wz1 = 1.0 - fz, fz
                if not DENSE[l]:
                    hx0 = x0.astype(jnp.uint32)
                    hx1 = x1.astype(jnp.uint32)
                    hy0 = y0.astype(jnp.uint32) * jnp.uint32(P1)
                    hy1 = y1.astype(jnp.uint32) * jnp.uint32(P1)
                    hz0 = z0.astype(jnp.uint32) * jnp.uint32(P2)
                    hz1 = z1.astype(jnp.uint32) * jnp.uint32(P2)
                for c in range(8):
                    cxb, cyb, czb = c & 1, (c >> 1) & 1, (c >> 2) & 1
                    w = ((wx1 if cxb else wx0) * (wy1 if cyb else wy0)) * (wz1 if czb else wz0)
                    if DENSE[l]:
                        s = res + 1
                        idx = (x1 if cxb else x0) + (y1 if cyb else y0) * s \
                            + (z1 if czb else z0) * (s * s)
                        if l >= N_STAGED:
                            idx = idx + (l * T)
                    else:
                        h = (hx1 if cxb else hx0) ^ (hy1 if cyb else hy0) ^ (hz1 if czb else hz0)
                        idx = (h & jnp.uint32(T - 1)).astype(jnp.int32) + (l * T)
                    idxrefs[l][pl.ds(c * LANES, LANES)] = lax.shift_right_logical(idx, 2)
                    colb[l, pl.ds(c * LANES, LANES)] = (idx & 3) * 2
                    wb[l, pl.ds(c * LANES, LANES)] = w

            copies = {}
            for l in range(N_STAGED, N_LEVELS):
                cp = pltpu.make_async_copy(table_hbm.at[idxrefs[l]],
                                           rowrefs[l - N_STAGED], sem)
                cp.start()
                copies[l] = cp

            hcol = (k % FLUSH_EVERY) * CH
            for l in range(N_LEVELS):
                acc0 = jnp.zeros((LANES,), jnp.float32)
                acc1 = jnp.zeros((LANES,), jnp.float32)
                if l < N_STAGED:
                    src = strefs[l]
                else:
                    copies[l].wait()
                    src = rowrefs[l - N_STAGED]
                for c in range(8):
                    w = wb[l, pl.ds(c * LANES, LANES)]
                    colv = colb[l, pl.ds(c * LANES, LANES)]
                    if l < N_STAGED:
                        rows = idxrefs[l][pl.ds(c * LANES, LANES)]
                    else:
                        rows = c * LANES + iota
                    f0 = plsc.load_gather(src, [rows, colv])
                    f1 = plsc.load_gather(src, [rows, colv + 1])
                    acc0 = acc0 + f0 * w
                    acc1 = acc1 + f1 * w
                hbuf[2 * l, pl.ds(hcol, CH)] = acc0
                hbuf[2 * l + 1, pl.ds(hcol, CH)] = acc1

            @pl.when(k % FLUSH_EVERY == FLUSH_EVERY - 1)
            def _():
                ob = pl.multiple_of(base + (k - (FLUSH_EVERY - 1)) * CH, HBUF)
                pltpu.sync_copy(hbuf, h_hbm.at[:, pl.ds(ob, HBUF)])

            return carry

        lax.fori_loop(0, nchunk, chunk_body, 0)

    return _encode_body


N_HALF = N_PTS // 2
PW_HALF = N_HALF // NW


def _make_encode(x_off, pw, npts):
    body = _make_encode_body(x_off, pw)
    mesh = plsc.VectorSubcoreMesh(core_axis_name="c", subcore_axis_name="s",
                                  num_cores=NC, num_subcores=NS)

    @jax.jit
    def enc(xt, table8):
        return pl.kernel(
            body,
            out_type=jax.ShapeDtypeStruct((ENC, npts), jnp.float32),
            mesh=mesh,
            compiler_params=pltpu.CompilerParams(needs_layout_passes=False,
                                                 use_tc_tiling_on_sc=False),
            scratch_types=(
                [
                    pltpu.VMEM((DIM, pw), jnp.float32),
                    pltpu.VMEM((N_LEVELS, 8 * LANES), jnp.float32),
                    pltpu.VMEM((N_LEVELS, 8 * LANES), jnp.int32),
                    pltpu.VMEM((ENC, HBUF), jnp.float32),
                    pltpu.SemaphoreType.DMA,
                ]
                + [pltpu.VMEM((8 * LANES,), jnp.int32) for _ in range(N_LEVELS)]
                + [pltpu.VMEM((8 * LANES, 8), jnp.float32)
                   for _ in range(N_LEVELS - N_STAGED)]
                + [pltpu.VMEM((STAGED_ROWS[l], 8), jnp.float32)
                   for l in range(N_STAGED)]
            ),
        )(xt, table8)

    return enc


_encode_h0 = _make_encode(0, PW_HALF, N_HALF)
_encode_h1 = _make_encode(N_HALF, PW_HALF, N_HALF)


BLK = 1024


def _mlp_body(h_ref, w0_ref, b0_ref, w1_ref, b1_ref, w2_ref, b2_ref, o_ref):
    h = h_ref[...]
    a = lax.dot_general(h, w0_ref[...], (((0,), (0,)), ((), ())),
                        preferred_element_type=jnp.float32)
    a = jnp.maximum(a + b0_ref[...], 0.0)
    a = jnp.dot(a, w1_ref[...], preferred_element_type=jnp.float32)
    a = jnp.maximum(a + b1_ref[...], 0.0)
    o_ref[...] = lax.dot_general(w2_ref[...], a, (((0,), (1,)), ((), ())),
                                 preferred_element_type=jnp.float32) + b2_ref[...]


@jax.jit
def _mlp(h, W0, b0, W1, b1, W2, b2):
    npts = h.shape[1]
    grid = (npts // BLK,)
    return pl.pallas_call(
        _mlp_body,
        grid=grid,
        in_specs=[
            pl.BlockSpec((ENC, BLK), lambda i: (0, i)),
            pl.BlockSpec((ENC, HIDDEN), lambda i: (0, 0)),
            pl.BlockSpec((1, HIDDEN), lambda i: (0, 0)),
            pl.BlockSpec((HIDDEN, HIDDEN), lambda i: (0, 0)),
            pl.BlockSpec((1, HIDDEN), lambda i: (0, 0)),
            pl.BlockSpec((HIDDEN, GRID_OUT), lambda i: (0, 0)),
            pl.BlockSpec((GRID_OUT, 1), lambda i: (0, 0)),
        ],
        out_specs=pl.BlockSpec((GRID_OUT, BLK), lambda i: (0, i)),
        out_shape=jax.ShapeDtypeStruct((GRID_OUT, npts), jnp.float32),
    )(h, W0, b0, W1, b1, W2, b2)


def kernel(x, table, W0, b0, W1, b1, W2, b2):
    xt = x.T
    tphys = (table.transpose(0, 2, 1)
             .reshape(N_LEVELS, F, T // 128, 128)
             .transpose(0, 2, 1, 3)
             .reshape(TBL_ELEMS))
    t8 = _relayout(tphys).reshape(TBL_ELEMS // 8, 8)
    b0r, b1r, b2r = b0.reshape(1, -1), b1.reshape(1, -1), b2.reshape(-1, 1)
    h1 = _encode_h0(xt, t8)
    o1 = _mlp(h1, W0, b0r, W1, b1r, W2, b2r)
    h2 = _encode_h1(xt, t8)
    o2 = _mlp(h2, W0, b0r, W1, b1r, W2, b2r)
    return jnp.concatenate([o1, o2], axis=1).T


# speedup vs baseline: 1.0954x; 1.0954x over previous
import functools
import numpy as np
import jax
import jax.numpy as jnp
from jax import lax
from jax.experimental import pallas as pl
from jax.experimental.pallas import tpu as pltpu
from jax.experimental.pallas import tpu_sc as plsc

N_LEVELS = 16
F = 2
BASE = 16
TARGET = 2048
LOG2_T = 19
T = 2 ** LOG2_T
DIM = 3
N_PTS = 131072
GRID_OUT = 64
HIDDEN = (GRID_OUT + 15) // 16 * 16
SCALE = np.exp2(np.log2(TARGET / BASE) / (N_LEVELS - 1))
RES = [int(np.floor(BASE * SCALE ** l)) for l in range(N_LEVELS)]
DENSE = [(r + 1) ** 3 <= T for r in RES]
P1, P2 = 2654435761, 805459861
ENC = N_LEVELS * F

NC, NS, LANES = 2, 16, 16
NW = NC * NS
PW = N_PTS // NW
CH = 16
NCHUNK = PW // CH
HBUF = 512
FLUSH_EVERY = HBUF // CH

TBL_ELEMS = N_LEVELS * T * F
RL_CHF = 16384
RL_SPAN = TBL_ELEMS // NW
RL_NCH = RL_SPAN // RL_CHF


def _relayout_body(tin_hbm, tout_hbm, binA, binB, boutA, boutB,
                   semiA, semiB, semoA, semoB):
    wid = lax.axis_index("s") * NC + lax.axis_index("c")
    base = wid * RL_SPAN
    iota = lax.iota(jnp.int32, LANES)

    def mk_in(k, buf, sem):
        return pltpu.make_async_copy(
            tin_hbm.at[pl.ds(base + k * RL_CHF, RL_CHF)], buf, sem)

    def mk_out(k, buf, sem):
        return pltpu.make_async_copy(
            buf, tout_hbm.at[pl.ds(base + k * RL_CHF, RL_CHF)], sem)

    def interleave(src, dst):
        def block(b, carry2):
            b256 = b * 256

            def eight(m, carry3):
                m16 = b256 + m * 16
                f0 = src[pl.ds(m16, LANES)]
                f1 = src[pl.ds(m16 + 128, LANES)]
                pos = b256 + m * 32 + 2 * iota
                plsc.store_scatter(dst, [pos], f0)
                plsc.store_scatter(dst, [pos + 1], f1)
                return carry3

            return lax.fori_loop(0, 8, eight, carry2)

        lax.fori_loop(0, RL_CHF // 256, block, 0)

    mk_in(0, binA, semiA).start()

    def pair(kk, carry):
        k0 = kk * 2
        mk_in(k0, binA, semiA).wait()
        mk_in(k0 + 1, binB, semiB).start()

        @pl.when(kk > 0)
        def _():
            mk_out(k0, boutA, semoA).wait()

        interleave(binA, boutA)
        mk_out(k0, boutA, semoA).start()

        mk_in(k0 + 1, binB, semiB).wait()

        @pl.when(kk + 1 < RL_NCH // 2)
        def _():
            mk_in(k0 + 2, binA, semiA).start()

        @pl.when(kk > 0)
        def _():
            mk_out(k0 + 1, boutB, semoB).wait()

        interleave(binB, boutB)
        mk_out(k0 + 1, boutB, semoB).start()
        return carry

    lax.fori_loop(0, RL_NCH // 2, pair, 0)
    mk_out(0, boutA, semoA).wait()
    mk_out(0, boutB, semoB).wait()


@jax.jit
def _relayout(tflat):
    mesh = plsc.VectorSubcoreMesh(core_axis_name="c", subcore_axis_name="s",
                                  num_cores=NC, num_subcores=NS)
    return pl.kernel(
        _relayout_body,
        out_type=jax.ShapeDtypeStruct((TBL_ELEMS,), jnp.float32),
        mesh=mesh,
        compiler_params=pltpu.CompilerParams(needs_layout_passes=False,
                                             use_tc_tiling_on_sc=False),
        scratch_types=[
            pltpu.VMEM((RL_CHF,), jnp.float32),
            pltpu.VMEM((RL_CHF,), jnp.float32),
            pltpu.VMEM((RL_CHF,), jnp.float32),
            pltpu.VMEM((RL_CHF,), jnp.float32),
            pltpu.SemaphoreType.DMA,
            pltpu.SemaphoreType.DMA,
            pltpu.SemaphoreType.DMA,
            pltpu.SemaphoreType.DMA,
        ],
    )(tflat)


N_STAGED = 2
STAGED_ROWS = [(RES[l] + 1) ** 3 * F // 8 + 8 for l in range(N_STAGED)]


def _make_encode_body(x_off, pw):
    nchunk = pw // CH

    def _encode_body(xt_hbm, table_hbm, h_hbm, xv, wb, colb, hbuf, sem,
                     *lvl_scratch):
        idxrefs = lvl_scratch[:N_LEVELS]
        rowrefs = lvl_scratch[N_LEVELS:2 * N_LEVELS - N_STAGED]
        strefs = lvl_scratch[2 * N_LEVELS - N_STAGED:]
        wid = lax.axis_index("s") * NC + lax.axis_index("c")
        base = wid * pw
        pltpu.sync_copy(xt_hbm.at[:, pl.ds(x_off + base, pw)], xv)
        for l in range(N_STAGED):
            pltpu.sync_copy(
                table_hbm.at[pl.ds(l * (T * F // 8), STAGED_ROWS[l])],
                strefs[l])
        iota = lax.iota(jnp.int32, LANES)

        def chunk_body(k, carry):
            off = k * CH
            xc = xv[0, pl.ds(off, CH)]
            yc = xv[1, pl.ds(off, CH)]
            zc = xv[2, pl.ds(off, CH)]

            for l in range(N_LEVELS):
                res = RES[l]
                resf = float(res)
                px = xc * resf
                py = yc * resf
                pz = zc * resf
                ix = px.astype(jnp.int32)
                iy = py.astype(jnp.int32)
                iz = pz.astype(jnp.int32)
                fx = px - ix.astype(jnp.float32)
                fy = py - iy.astype(jnp.float32)
                fz = pz - iz.astype(jnp.float32)
                x0, x1 = ix, jnp.minimum(ix + 1, res)
                y0, y1 = iy, jnp.minimum(iy + 1, res)
                z0, z1 = iz, jnp.minimum(iz + 1, res)
                wx0, wx1 = 1.0 - fx, fx
                wy0, wy1 = 1.0 - fy, fy
                wz0, wz1 = 1.0 - fz, fz
                if not DENSE[l]:
                    hx0 = x0.astype(jnp.uint32)
                    hx1 = x1.astype(jnp.uint32)
                    hy0 = y0.astype(jnp.uint32) * jnp.uint32(P1)
                    hy1 = y1.astype(jnp.uint32) * jnp.uint32(P1)
                    hz0 = z0.astype(jnp.uint32) * jnp.uint32(P2)
                    hz1 = z1.astype(jnp.uint32) * jnp.uint32(P2)
                for c in range(8):
                    cxb, cyb, czb = c & 1, (c >> 1) & 1, (c >> 2) & 1
                    w = ((wx1 if cxb else wx0) * (wy1 if cyb else wy0)) * (wz1 if czb else wz0)
                    if DENSE[l]:
                        s = res + 1
                        idx = (x1 if cxb else x0) + (y1 if cyb else y0) * s \
                            + (z1 if czb else z0) * (s * s)
                        if l >= N_STAGED:
                            idx = idx + (l * T)
                    else:
                        h = (hx1 if cxb else hx0) ^ (hy1 if cyb else hy0) ^ (hz1 if czb else hz0)
                        idx = (h & jnp.uint32(T - 1)).astype(jnp.int32) + (l * T)
                    idxrefs[l][pl.ds(c * LANES, LANES)] = lax.shift_right_logical(idx, 2)
                    colb[l, pl.ds(c * LANES, LANES)] = (idx & 3) * 2
                    wb[l, pl.ds(c * LANES, LANES)] = w

            copies = {}
            for l in range(N_STAGED, N_LEVELS):
                cp = pltpu.make_async_copy(table_hbm.at[idxrefs[l]],
                                           rowrefs[l - N_STAGED], sem)
                cp.start()
                copies[l] = cp

            hcol = (k % FLUSH_EVERY) * CH
            for l in range(N_LEVELS):
                acc0 = jnp.zeros((LANES,), jnp.float32)
                acc1 = jnp.zeros((LANES,), jnp.float32)
                if l < N_STAGED:
                    src = strefs[l]
                else:
                    copies[l].wait()
                    src = rowrefs[l - N_STAGED]
                for c in range(8):
                    w = wb[l, pl.ds(c * LANES, LANES)]
                    colv = colb[l, pl.ds(c * LANES, LANES)]
                    if l < N_STAGED:
                        rows = idxrefs[l][pl.ds(c * LANES, LANES)]
                    else:
                        rows = c * LANES + iota
                    f0 = plsc.load_gather(src, [rows, colv])
                    f1 = plsc.load_gather(src, [rows, colv + 1])
                    acc0 = acc0 + f0 * w
                    acc1 = acc1 + f1 * w
                hbuf[2 * l, pl.ds(hcol, CH)] = acc0
                hbuf[2 * l + 1, pl.ds(hcol, CH)] = acc1

            @pl.when(k % FLUSH_EVERY == FLUSH_EVERY - 1)
            def _():
                ob = pl.multiple_of(base + (k - (FLUSH_EVERY - 1)) * CH, HBUF)
                pltpu.sync_copy(hbuf, h_hbm.at[:, pl.ds(ob, HBUF)])

            return carry

        lax.fori_loop(0, nchunk, chunk_body, 0)

    return _encode_body


N_HALF = N_PTS // 2
PW_HALF = N_HALF // NW


def _make_encode(x_off, pw, npts):
    body = _make_encode_body(x_off, pw)
    mesh = plsc.VectorSubcoreMesh(core_axis_name="c", subcore_axis_name="s",
                                  num_cores=NC, num_subcores=NS)

    @jax.jit
    def enc(xt, table8):
        return pl.kernel(
            body,
            out_type=jax.ShapeDtypeStruct((ENC, npts), jnp.float32),
            mesh=mesh,
            compiler_params=pltpu.CompilerParams(needs_layout_passes=False,
                                                 use_tc_tiling_on_sc=False),
            scratch_types=(
                [
                    pltpu.VMEM((DIM, pw), jnp.float32),
                    pltpu.VMEM((N_LEVELS, 8 * LANES), jnp.float32),
                    pltpu.VMEM((N_LEVELS, 8 * LANES), jnp.int32),
                    pltpu.VMEM((ENC, HBUF), jnp.float32),
                    pltpu.SemaphoreType.DMA,
                ]
                + [pltpu.VMEM((8 * LANES,), jnp.int32) for _ in range(N_LEVELS)]
                + [pltpu.VMEM((8 * LANES, 8), jnp.float32)
                   for _ in range(N_LEVELS - N_STAGED)]
                + [pltpu.VMEM((STAGED_ROWS[l], 8), jnp.float32)
                   for l in range(N_STAGED)]
            ),
        )(xt, table8)

    return enc


_encode_full = _make_encode(0, PW, N_PTS)


BLK = 1024


def _mlp_body(h_ref, w0_ref, b0_ref, w1_ref, b1_ref, w2_ref, b2_ref, o_ref):
    h = h_ref[...]
    a = lax.dot_general(h, w0_ref[...], (((0,), (0,)), ((), ())),
                        preferred_element_type=jnp.float32)
    a = jnp.maximum(a + b0_ref[...], 0.0)
    a = jnp.dot(a, w1_ref[...], preferred_element_type=jnp.float32)
    a = jnp.maximum(a + b1_ref[...], 0.0)
    o_ref[...] = lax.dot_general(w2_ref[...], a, (((0,), (1,)), ((), ())),
                                 preferred_element_type=jnp.float32) + b2_ref[...]


@jax.jit
def _mlp(h, W0, b0, W1, b1, W2, b2):
    npts = h.shape[1]
    grid = (npts // BLK,)
    return pl.pallas_call(
        _mlp_body,
        grid=grid,
        in_specs=[
            pl.BlockSpec((ENC, BLK), lambda i: (0, i)),
            pl.BlockSpec((ENC, HIDDEN), lambda i: (0, 0)),
            pl.BlockSpec((1, HIDDEN), lambda i: (0, 0)),
            pl.BlockSpec((HIDDEN, HIDDEN), lambda i: (0, 0)),
            pl.BlockSpec((1, HIDDEN), lambda i: (0, 0)),
            pl.BlockSpec((HIDDEN, GRID_OUT), lambda i: (0, 0)),
            pl.BlockSpec((GRID_OUT, 1), lambda i: (0, 0)),
        ],
        out_specs=pl.BlockSpec((GRID_OUT, BLK), lambda i: (0, i)),
        out_shape=jax.ShapeDtypeStruct((GRID_OUT, npts), jnp.float32),
    )(h, W0, b0, W1, b1, W2, b2)


def kernel(x, table, W0, b0, W1, b1, W2, b2):
    xt = x.T
    tphys = (table.transpose(0, 2, 1)
             .reshape(N_LEVELS, F, T // 128, 128)
             .transpose(0, 2, 1, 3)
             .reshape(TBL_ELEMS))
    t8 = _relayout(tphys).reshape(TBL_ELEMS // 8, 8)
    h = _encode_full(xt, t8)
    out_t = _mlp(h, W0, b0.reshape(1, -1), W1, b1.reshape(1, -1),
                 W2, b2.reshape(-1, 1))
    return out_t.T


# speedup vs baseline: 1.3365x; 1.2201x over previous
import functools
import numpy as np
import jax
import jax.numpy as jnp
from jax import lax
from jax.experimental import pallas as pl
from jax.experimental.pallas import tpu as pltpu
from jax.experimental.pallas import tpu_sc as plsc

N_LEVELS = 16
F = 2
BASE = 16
TARGET = 2048
LOG2_T = 19
T = 2 ** LOG2_T
DIM = 3
N_PTS = 131072
GRID_OUT = 64
HIDDEN = (GRID_OUT + 15) // 16 * 16
SCALE = np.exp2(np.log2(TARGET / BASE) / (N_LEVELS - 1))
RES = [int(np.floor(BASE * SCALE ** l)) for l in range(N_LEVELS)]
DENSE = [(r + 1) ** 3 <= T for r in RES]
P1, P2 = 2654435761, 805459861
ENC = N_LEVELS * F

NC, NS, LANES = 2, 16, 16
NW = NC * NS
PW = N_PTS // NW
CH = 16
NCHUNK = PW // CH
HBUF = 512
FLUSH_EVERY = HBUF // CH

TBL_ELEMS = N_LEVELS * T * F
RL_CHF = 16384
RL_SPAN = TBL_ELEMS // NW
RL_NCH = RL_SPAN // RL_CHF


def _relayout_body(tin_hbm, tout_hbm, binA, binB, boutA, boutB,
                   semiA, semiB, semoA, semoB):
    wid = lax.axis_index("s") * NC + lax.axis_index("c")
    base = wid * RL_SPAN
    iota = lax.iota(jnp.int32, LANES)

    def mk_in(k, buf, sem):
        return pltpu.make_async_copy(
            tin_hbm.at[pl.ds(base + k * RL_CHF, RL_CHF)], buf, sem)

    def mk_out(k, buf, sem):
        return pltpu.make_async_copy(
            buf, tout_hbm.at[pl.ds(base + k * RL_CHF, RL_CHF)], sem)

    def interleave(src, dst):
        def block(b, carry2):
            b256 = b * 256

            def eight(m, carry3):
                m16 = b256 + m * 16
                f0 = src[pl.ds(m16, LANES)]
                f1 = src[pl.ds(m16 + 128, LANES)]
                pos = b256 + m * 32 + 2 * iota
                plsc.store_scatter(dst, [pos], f0)
                plsc.store_scatter(dst, [pos + 1], f1)
                return carry3

            return lax.fori_loop(0, 8, eight, carry2)

        lax.fori_loop(0, RL_CHF // 256, block, 0)

    mk_in(0, binA, semiA).start()

    def pair(kk, carry):
        k0 = kk * 2
        mk_in(k0, binA, semiA).wait()
        mk_in(k0 + 1, binB, semiB).start()

        @pl.when(kk > 0)
        def _():
            mk_out(k0, boutA, semoA).wait()

        interleave(binA, boutA)
        mk_out(k0, boutA, semoA).start()

        mk_in(k0 + 1, binB, semiB).wait()

        @pl.when(kk + 1 < RL_NCH // 2)
        def _():
            mk_in(k0 + 2, binA, semiA).start()

        @pl.when(kk > 0)
        def _():
            mk_out(k0 + 1, boutB, semoB).wait()

        interleave(binB, boutB)
        mk_out(k0 + 1, boutB, semoB).start()
        return carry

    lax.fori_loop(0, RL_NCH // 2, pair, 0)
    mk_out(0, boutA, semoA).wait()
    mk_out(0, boutB, semoB).wait()


@jax.jit
def _relayout(tflat):
    mesh = plsc.VectorSubcoreMesh(core_axis_name="c", subcore_axis_name="s",
                                  num_cores=NC, num_subcores=NS)
    return pl.kernel(
        _relayout_body,
        out_type=jax.ShapeDtypeStruct((TBL_ELEMS,), jnp.float32),
        mesh=mesh,
        compiler_params=pltpu.CompilerParams(needs_layout_passes=False,
                                             use_tc_tiling_on_sc=False),
        scratch_types=[
            pltpu.VMEM((RL_CHF,), jnp.float32),
            pltpu.VMEM((RL_CHF,), jnp.float32),
            pltpu.VMEM((RL_CHF,), jnp.float32),
            pltpu.VMEM((RL_CHF,), jnp.float32),
            pltpu.SemaphoreType.DMA,
            pltpu.SemaphoreType.DMA,
            pltpu.SemaphoreType.DMA,
            pltpu.SemaphoreType.DMA,
        ],
    )(tflat)


N_STAGED = 2
STAGED_ROWS = [(RES[l] + 1) ** 3 * F // 8 + 8 for l in range(N_STAGED)]


def _make_encode_body(x_off, pw):
    nchunk = pw // CH

    def _encode_body(xt_hbm, table_hbm, h_hbm, xv, wb, colb, hbuf,
                     semA, semB, *lvl_scratch):
        nl = N_LEVELS
        idxsets = (lvl_scratch[:nl], lvl_scratch[nl:2 * nl])
        nr = nl - N_STAGED
        rowsets = (lvl_scratch[2 * nl:2 * nl + nr],
                   lvl_scratch[2 * nl + nr:2 * nl + 2 * nr])
        strefs = lvl_scratch[2 * nl + 2 * nr:]
        sems = (semA, semB)
        wid = lax.axis_index("s") * NC + lax.axis_index("c")
        base = wid * pw
        pltpu.sync_copy(xt_hbm.at[:, pl.ds(x_off + base, pw)], xv)
        for l in range(N_STAGED):
            pltpu.sync_copy(
                table_hbm.at[pl.ds(l * (T * F // 8), STAGED_ROWS[l])],
                strefs[l])
        iota = lax.iota(jnp.int32, LANES)

        def phase_a(k, si):
            idxrefs = idxsets[si]
            off = k * CH
            xc = xv[0, pl.ds(off, CH)]
            yc = xv[1, pl.ds(off, CH)]
            zc = xv[2, pl.ds(off, CH)]
            for l in range(N_LEVELS):
                res = RES[l]
                resf = float(res)
                px = xc * resf
                py = yc * resf
                pz = zc * resf
                ix = px.astype(jnp.int32)
                iy = py.astype(jnp.int32)
                iz = pz.astype(jnp.int32)
                fx = px - ix.astype(jnp.float32)
                fy = py - iy.astype(jnp.float32)
                fz = pz - iz.astype(jnp.float32)
                x0, x1 = ix, jnp.minimum(ix + 1, res)
                y0, y1 = iy, jnp.minimum(iy + 1, res)
                z0, z1 = iz, jnp.minimum(iz + 1, res)
                wx0, wx1 = 1.0 - fx, fx
                wy0, wy1 = 1.0 - fy, fy
                wz0, wz1 = 1.0 - fz, fz
                if not DENSE[l]:
                    hx0 = x0.astype(jnp.uint32)
                    hx1 = x1.astype(jnp.uint32)
                    hy0 = y0.astype(jnp.uint32) * jnp.uint32(P1)
                    hy1 = y1.astype(jnp.uint32) * jnp.uint32(P1)
                    hz0 = z0.astype(jnp.uint32) * jnp.uint32(P2)
                    hz1 = z1.astype(jnp.uint32) * jnp.uint32(P2)
                for c in range(8):
                    cxb, cyb, czb = c & 1, (c >> 1) & 1, (c >> 2) & 1
                    w = ((wx1 if cxb else wx0) * (wy1 if cyb else wy0)) * (wz1 if czb else wz0)
                    if DENSE[l]:
                        s = res + 1
                        idx = (x1 if cxb else x0) + (y1 if cyb else y0) * s \
                            + (z1 if czb else z0) * (s * s)
                        if l >= N_STAGED:
                            idx = idx + (l * T)
                    else:
                        h = (hx1 if cxb else hx0) ^ (hy1 if cyb else hy0) ^ (hz1 if czb else hz0)
                        idx = (h & jnp.uint32(T - 1)).astype(jnp.int32) + (l * T)
                    idxrefs[l][pl.ds(c * LANES, LANES)] = lax.shift_right_logical(idx, 2)
                    colb[si * N_LEVELS + l, pl.ds(c * LANES, LANES)] = (idx & 3) * 2
                    wb[si * N_LEVELS + l, pl.ds(c * LANES, LANES)] = w
            for l in range(N_STAGED, N_LEVELS):
                pltpu.make_async_copy(table_hbm.at[idxrefs[l]],
                                      rowsets[si][l - N_STAGED],
                                      sems[si]).start()

        def phase_b(k, si):
            idxrefs = idxsets[si]
            hcol = (k % FLUSH_EVERY) * CH
            for l in range(N_LEVELS):
                acc0 = jnp.zeros((LANES,), jnp.float32)
                acc1 = jnp.zeros((LANES,), jnp.float32)
                if l < N_STAGED:
                    src = strefs[l]
                else:
                    pltpu.make_async_copy(table_hbm.at[idxrefs[l]],
                                          rowsets[si][l - N_STAGED],
                                          sems[si]).wait()
                    src = rowsets[si][l - N_STAGED]
                for c in range(8):
                    w = wb[si * N_LEVELS + l, pl.ds(c * LANES, LANES)]
                    colv = colb[si * N_LEVELS + l, pl.ds(c * LANES, LANES)]
                    if l < N_STAGED:
                        rows = idxrefs[l][pl.ds(c * LANES, LANES)]
                    else:
                        rows = c * LANES + iota
                    f0 = plsc.load_gather(src, [rows, colv])
                    f1 = plsc.load_gather(src, [rows, colv + 1])
                    acc0 = acc0 + f0 * w
                    acc1 = acc1 + f1 * w
                hbuf[2 * l, pl.ds(hcol, CH)] = acc0
                hbuf[2 * l + 1, pl.ds(hcol, CH)] = acc1

            @pl.when(k % FLUSH_EVERY == FLUSH_EVERY - 1)
            def _():
                ob = pl.multiple_of(base + (k - (FLUSH_EVERY - 1)) * CH, HBUF)
                pltpu.sync_copy(hbuf, h_hbm.at[:, pl.ds(ob, HBUF)])

        phase_a(0, 0)

        def pair(kk, carry):
            k0 = 2 * kk
            phase_a(k0 + 1, 1)
            phase_b(k0, 0)

            @pl.when(kk + 1 < nchunk // 2)
            def _():
                phase_a(k0 + 2, 0)

            phase_b(k0 + 1, 1)
            return carry

        lax.fori_loop(0, nchunk // 2, pair, 0)

    return _encode_body


N_HALF = N_PTS // 2
PW_HALF = N_HALF // NW


def _make_encode(x_off, pw, npts):
    body = _make_encode_body(x_off, pw)
    mesh = plsc.VectorSubcoreMesh(core_axis_name="c", subcore_axis_name="s",
                                  num_cores=NC, num_subcores=NS)

    @jax.jit
    def enc(xt, table8):
        return pl.kernel(
            body,
            out_type=jax.ShapeDtypeStruct((ENC, npts), jnp.float32),
            mesh=mesh,
            compiler_params=pltpu.CompilerParams(needs_layout_passes=False,
                                                 use_tc_tiling_on_sc=False),
            scratch_types=(
                [
                    pltpu.VMEM((DIM, pw), jnp.float32),
                    pltpu.VMEM((2 * N_LEVELS, 8 * LANES), jnp.float32),
                    pltpu.VMEM((2 * N_LEVELS, 8 * LANES), jnp.int32),
                    pltpu.VMEM((ENC, HBUF), jnp.float32),
                    pltpu.SemaphoreType.DMA,
                    pltpu.SemaphoreType.DMA,
                ]
                + [pltpu.VMEM((8 * LANES,), jnp.int32)
                   for _ in range(2 * N_LEVELS)]
                + [pltpu.VMEM((8 * LANES, 8), jnp.float32)
                   for _ in range(2 * (N_LEVELS - N_STAGED))]
                + [pltpu.VMEM((STAGED_ROWS[l], 8), jnp.float32)
                   for l in range(N_STAGED)]
            ),
        )(xt, table8)

    return enc


_encode_full = _make_encode(0, PW, N_PTS)


BLK = 1024


def _mlp_body(h_ref, w0_ref, b0_ref, w1_ref, b1_ref, w2_ref, b2_ref, o_ref):
    h = h_ref[...]
    a = lax.dot_general(h, w0_ref[...], (((0,), (0,)), ((), ())),
                        preferred_element_type=jnp.float32)
    a = jnp.maximum(a + b0_ref[...], 0.0)
    a = jnp.dot(a, w1_ref[...], preferred_element_type=jnp.float32)
    a = jnp.maximum(a + b1_ref[...], 0.0)
    o_ref[...] = lax.dot_general(w2_ref[...], a, (((0,), (1,)), ((), ())),
                                 preferred_element_type=jnp.float32) + b2_ref[...]


@jax.jit
def _mlp(h, W0, b0, W1, b1, W2, b2):
    npts = h.shape[1]
    grid = (npts // BLK,)
    return pl.pallas_call(
        _mlp_body,
        grid=grid,
        in_specs=[
            pl.BlockSpec((ENC, BLK), lambda i: (0, i)),
            pl.BlockSpec((ENC, HIDDEN), lambda i: (0, 0)),
            pl.BlockSpec((1, HIDDEN), lambda i: (0, 0)),
            pl.BlockSpec((HIDDEN, HIDDEN), lambda i: (0, 0)),
            pl.BlockSpec((1, HIDDEN), lambda i: (0, 0)),
            pl.BlockSpec((HIDDEN, GRID_OUT), lambda i: (0, 0)),
            pl.BlockSpec((GRID_OUT, 1), lambda i: (0, 0)),
        ],
        out_specs=pl.BlockSpec((GRID_OUT, BLK), lambda i: (0, i)),
        out_shape=jax.ShapeDtypeStruct((GRID_OUT, npts), jnp.float32),
    )(h, W0, b0, W1, b1, W2, b2)


def kernel(x, table, W0, b0, W1, b1, W2, b2):
    xt = x.T
    tphys = (table.transpose(0, 2, 1)
             .reshape(N_LEVELS, F, T // 128, 128)
             .transpose(0, 2, 1, 3)
             .reshape(TBL_ELEMS))
    t8 = _relayout(tphys).reshape(TBL_ELEMS // 8, 8)
    h = _encode_full(xt, t8)
    out_t = _mlp(h, W0, b0.reshape(1, -1), W1, b1.reshape(1, -1),
                 W2, b2.reshape(-1, 1))
    return out_t.T


# speedup vs baseline: 1.3499x; 1.0100x over previous
import functools
import numpy as np
import jax
import jax.numpy as jnp
from jax import lax
from jax.experimental import pallas as pl
from jax.experimental.pallas import tpu as pltpu
from jax.experimental.pallas import tpu_sc as plsc

N_LEVELS = 16
F = 2
BASE = 16
TARGET = 2048
LOG2_T = 19
T = 2 ** LOG2_T
DIM = 3
N_PTS = 131072
GRID_OUT = 64
HIDDEN = (GRID_OUT + 15) // 16 * 16
SCALE = np.exp2(np.log2(TARGET / BASE) / (N_LEVELS - 1))
RES = [int(np.floor(BASE * SCALE ** l)) for l in range(N_LEVELS)]
DENSE = [(r + 1) ** 3 <= T for r in RES]
P1, P2 = 2654435761, 805459861
ENC = N_LEVELS * F

NC, NS, LANES = 2, 16, 16
NW = NC * NS
PW = N_PTS // NW
CH = 16
NCHUNK = PW // CH
HBUF = 512
FLUSH_EVERY = HBUF // CH

TBL_ELEMS = N_LEVELS * T * F
RL_CHF = 16384
RL_SPAN = TBL_ELEMS // NW
RL_NCH = RL_SPAN // RL_CHF


def _relayout_body(tin_hbm, tout_hbm, binA, binB, boutA, boutB,
                   semiA, semiB, semoA, semoB):
    wid = lax.axis_index("s") * NC + lax.axis_index("c")
    base = wid * RL_SPAN
    iota = lax.iota(jnp.int32, LANES)

    def mk_in(k, buf, sem):
        return pltpu.make_async_copy(
            tin_hbm.at[pl.ds(base + k * RL_CHF, RL_CHF)], buf, sem)

    def mk_out(k, buf, sem):
        return pltpu.make_async_copy(
            buf, tout_hbm.at[pl.ds(base + k * RL_CHF, RL_CHF)], sem)

    def interleave(src, dst):
        def block(b, carry2):
            b256 = b * 256

            def eight(m, carry3):
                m16 = b256 + m * 16
                f0 = src[pl.ds(m16, LANES)]
                f1 = src[pl.ds(m16 + 128, LANES)]
                pos = b256 + m * 32 + 2 * iota
                plsc.store_scatter(dst, [pos], f0)
                plsc.store_scatter(dst, [pos + 1], f1)
                return carry3

            return lax.fori_loop(0, 8, eight, carry2)

        lax.fori_loop(0, RL_CHF // 256, block, 0)

    mk_in(0, binA, semiA).start()

    def pair(kk, carry):
        k0 = kk * 2
        mk_in(k0, binA, semiA).wait()
        mk_in(k0 + 1, binB, semiB).start()

        @pl.when(kk > 0)
        def _():
            mk_out(k0, boutA, semoA).wait()

        interleave(binA, boutA)
        mk_out(k0, boutA, semoA).start()

        mk_in(k0 + 1, binB, semiB).wait()

        @pl.when(kk + 1 < RL_NCH // 2)
        def _():
            mk_in(k0 + 2, binA, semiA).start()

        @pl.when(kk > 0)
        def _():
            mk_out(k0 + 1, boutB, semoB).wait()

        interleave(binB, boutB)
        mk_out(k0 + 1, boutB, semoB).start()
        return carry

    lax.fori_loop(0, RL_NCH // 2, pair, 0)
    mk_out(0, boutA, semoA).wait()
    mk_out(0, boutB, semoB).wait()


@jax.jit
def _relayout(tflat):
    mesh = plsc.VectorSubcoreMesh(core_axis_name="c", subcore_axis_name="s",
                                  num_cores=NC, num_subcores=NS)
    return pl.kernel(
        _relayout_body,
        out_type=jax.ShapeDtypeStruct((TBL_ELEMS,), jnp.float32),
        mesh=mesh,
        compiler_params=pltpu.CompilerParams(needs_layout_passes=False,
                                             use_tc_tiling_on_sc=False),
        scratch_types=[
            pltpu.VMEM((RL_CHF,), jnp.float32),
            pltpu.VMEM((RL_CHF,), jnp.float32),
            pltpu.VMEM((RL_CHF,), jnp.float32),
            pltpu.VMEM((RL_CHF,), jnp.float32),
            pltpu.SemaphoreType.DMA,
            pltpu.SemaphoreType.DMA,
            pltpu.SemaphoreType.DMA,
            pltpu.SemaphoreType.DMA,
        ],
    )(tflat)


N_STAGED = 2
STAGED_ROWS = [(RES[l] + 1) ** 3 * F // 8 + 8 for l in range(N_STAGED)]


def _make_encode_body(x_off, pw):
    nchunk = pw // CH

    def _encode_body(xt_hbm, table_hbm, h_hbm, xv, wb, colb, hbuf,
                     semA, semB, *lvl_scratch):
        nl = N_LEVELS
        idxsets = (lvl_scratch[:nl], lvl_scratch[nl:2 * nl])
        nr = nl - N_STAGED
        rowsets = (lvl_scratch[2 * nl:2 * nl + nr],
                   lvl_scratch[2 * nl + nr:2 * nl + 2 * nr])
        strefs = lvl_scratch[2 * nl + 2 * nr:]
        sems = (semA, semB)
        wid = lax.axis_index("s") * NC + lax.axis_index("c")
        base = wid * pw
        pltpu.sync_copy(xt_hbm.at[:, pl.ds(x_off + base, pw)], xv)
        for l in range(N_STAGED):
            pltpu.sync_copy(
                table_hbm.at[pl.ds(l * (T * F // 8), STAGED_ROWS[l])],
                strefs[l])
        iota = lax.iota(jnp.int32, LANES)

        def phase_a(k, si):
            idxrefs = idxsets[si]
            off = k * CH
            xc = xv[0, pl.ds(off, CH)]
            yc = xv[1, pl.ds(off, CH)]
            zc = xv[2, pl.ds(off, CH)]
            for l in range(N_LEVELS):
                res = RES[l]
                resf = float(res)
                px = xc * resf
                py = yc * resf
                pz = zc * resf
                ix = px.astype(jnp.int32)
                iy = py.astype(jnp.int32)
                iz = pz.astype(jnp.int32)
                fx = px - ix.astype(jnp.float32)
                fy = py - iy.astype(jnp.float32)
                fz = pz - iz.astype(jnp.float32)
                x0, x1 = ix, jnp.minimum(ix + 1, res)
                y0, y1 = iy, jnp.minimum(iy + 1, res)
                z0, z1 = iz, jnp.minimum(iz + 1, res)
                wx0, wx1 = 1.0 - fx, fx
                wy0, wy1 = 1.0 - fy, fy
                wz0, wz1 = 1.0 - fz, fz
                if not DENSE[l]:
                    hx0 = x0.astype(jnp.uint32)
                    hx1 = x1.astype(jnp.uint32)
                    hy0 = y0.astype(jnp.uint32) * jnp.uint32(P1)
                    hy1 = y1.astype(jnp.uint32) * jnp.uint32(P1)
                    hz0 = z0.astype(jnp.uint32) * jnp.uint32(P2)
                    hz1 = z1.astype(jnp.uint32) * jnp.uint32(P2)
                for c in range(8):
                    cxb, cyb, czb = c & 1, (c >> 1) & 1, (c >> 2) & 1
                    w = ((wx1 if cxb else wx0) * (wy1 if cyb else wy0)) * (wz1 if czb else wz0)
                    if DENSE[l]:
                        s = res + 1
                        idx = (x1 if cxb else x0) + (y1 if cyb else y0) * s \
                            + (z1 if czb else z0) * (s * s)
                        if l >= N_STAGED:
                            idx = idx + (l * T)
                    else:
                        h = (hx1 if cxb else hx0) ^ (hy1 if cyb else hy0) ^ (hz1 if czb else hz0)
                        idx = (h & jnp.uint32(T - 1)).astype(jnp.int32) + (l * T)
                    idxrefs[l][pl.ds(c * LANES, LANES)] = lax.shift_right_logical(idx, 2)
                    colb[si * N_LEVELS + l, pl.ds(c * LANES, LANES)] = (idx & 3) * 2
                    wb[si * N_LEVELS + l, pl.ds(c * LANES, LANES)] = w
            for l in range(N_STAGED, N_LEVELS):
                pltpu.make_async_copy(table_hbm.at[idxrefs[l]],
                                      rowsets[si][l - N_STAGED],
                                      sems[si]).start()

        def phase_b(k, si):
            idxrefs = idxsets[si]
            hcol = (k % FLUSH_EVERY) * CH
            for l in range(N_LEVELS):
                acc0 = jnp.zeros((LANES,), jnp.float32)
                acc1 = jnp.zeros((LANES,), jnp.float32)
                if l < N_STAGED:
                    src = strefs[l]
                else:
                    pltpu.make_async_copy(table_hbm.at[idxrefs[l]],
                                          rowsets[si][l - N_STAGED],
                                          sems[si]).wait()
                    src = rowsets[si][l - N_STAGED]
                for c in range(8):
                    w = wb[si * N_LEVELS + l, pl.ds(c * LANES, LANES)]
                    colv = colb[si * N_LEVELS + l, pl.ds(c * LANES, LANES)]
                    if l < N_STAGED:
                        rows = idxrefs[l][pl.ds(c * LANES, LANES)]
                    else:
                        rows = c * LANES + iota
                    f0 = plsc.load_gather(src, [rows, colv])
                    f1 = plsc.load_gather(src, [rows, colv + 1])
                    acc0 = acc0 + f0 * w
                    acc1 = acc1 + f1 * w
                rb, rr = (2 * l) // 8, (2 * l) % 8
                cb_loc = hcol // 128
                cc = hcol % 128
                hbuf[rb, cb_loc, rr, pl.ds(cc, CH)] = acc0
                hbuf[rb, cb_loc, rr + 1, pl.ds(cc, CH)] = acc1

            @pl.when(k % FLUSH_EVERY == FLUSH_EVERY - 1)
            def _():
                ob = pl.multiple_of(base + (k - (FLUSH_EVERY - 1)) * CH, HBUF)
                colblk0 = ob // 128
                for rb in range(ENC // 8):
                    pltpu.sync_copy(
                        hbuf.at[rb],
                        h_hbm.at[pl.ds(rb * (pw * NW // 128) + colblk0,
                                       HBUF // 128)])

        phase_a(0, 0)

        def pair(kk, carry):
            k0 = 2 * kk
            phase_a(k0 + 1, 1)
            phase_b(k0, 0)

            @pl.when(kk + 1 < nchunk // 2)
            def _():
                phase_a(k0 + 2, 0)

            phase_b(k0 + 1, 1)
            return carry

        lax.fori_loop(0, nchunk // 2, pair, 0)

    return _encode_body


N_HALF = N_PTS // 2
PW_HALF = N_HALF // NW


def _make_encode(x_off, pw, npts):
    body = _make_encode_body(x_off, pw)
    mesh = plsc.VectorSubcoreMesh(core_axis_name="c", subcore_axis_name="s",
                                  num_cores=NC, num_subcores=NS)

    @jax.jit
    def enc(xt, table8):
        return pl.kernel(
            body,
            out_type=jax.ShapeDtypeStruct((npts // 32, 8, 128), jnp.float32),
            mesh=mesh,
            compiler_params=pltpu.CompilerParams(needs_layout_passes=False,
                                                 use_tc_tiling_on_sc=False),
            scratch_types=(
                [
                    pltpu.VMEM((DIM, pw), jnp.float32),
                    pltpu.VMEM((2 * N_LEVELS, 8 * LANES), jnp.float32),
                    pltpu.VMEM((2 * N_LEVELS, 8 * LANES), jnp.int32),
                    pltpu.VMEM((ENC // 8, HBUF // 128, 8, 128), jnp.float32),
                    pltpu.SemaphoreType.DMA,
                    pltpu.SemaphoreType.DMA,
                ]
                + [pltpu.VMEM((8 * LANES,), jnp.int32)
                   for _ in range(2 * N_LEVELS)]
                + [pltpu.VMEM((8 * LANES, 8), jnp.float32)
                   for _ in range(2 * (N_LEVELS - N_STAGED))]
                + [pltpu.VMEM((STAGED_ROWS[l], 8), jnp.float32)
                   for l in range(N_STAGED)]
            ),
        )(xt, table8)

    return enc


_encode_full = _make_encode(0, PW, N_PTS)


BLK = 1024


def _mlp_body(h_ref, w0_ref, b0_ref, w1_ref, b1_ref, w2_ref, b2_ref, o_ref):
    h = h_ref[...]
    a = lax.dot_general(h, w0_ref[...], (((0,), (0,)), ((), ())),
                        preferred_element_type=jnp.float32)
    a = jnp.maximum(a + b0_ref[...], 0.0)
    a = jnp.dot(a, w1_ref[...], preferred_element_type=jnp.float32)
    a = jnp.maximum(a + b1_ref[...], 0.0)
    o_ref[...] = lax.dot_general(w2_ref[...], a, (((0,), (1,)), ((), ())),
                                 preferred_element_type=jnp.float32) + b2_ref[...]


@jax.jit
def _mlp(h, W0, b0, W1, b1, W2, b2):
    npts = h.shape[1]
    grid = (npts // BLK,)
    return pl.pallas_call(
        _mlp_body,
        grid=grid,
        in_specs=[
            pl.BlockSpec((ENC, BLK), lambda i: (0, i)),
            pl.BlockSpec((ENC, HIDDEN), lambda i: (0, 0)),
            pl.BlockSpec((1, HIDDEN), lambda i: (0, 0)),
            pl.BlockSpec((HIDDEN, HIDDEN), lambda i: (0, 0)),
            pl.BlockSpec((1, HIDDEN), lambda i: (0, 0)),
            pl.BlockSpec((HIDDEN, GRID_OUT), lambda i: (0, 0)),
            pl.BlockSpec((GRID_OUT, 1), lambda i: (0, 0)),
        ],
        out_specs=pl.BlockSpec((GRID_OUT, BLK), lambda i: (0, i)),
        out_shape=jax.ShapeDtypeStruct((GRID_OUT, npts), jnp.float32),
    )(h, W0, b0, W1, b1, W2, b2)


def kernel(x, table, W0, b0, W1, b1, W2, b2):
    xt = x.T
    tphys = (table.transpose(0, 2, 1)
             .reshape(N_LEVELS, F, T // 128, 128)
             .transpose(0, 2, 1, 3)
             .reshape(TBL_ELEMS))
    t8 = _relayout(tphys).reshape(TBL_ELEMS // 8, 8)
    h4 = _encode_full(xt, t8)
    h = (h4.reshape(ENC // 8, N_PTS // 128, 8, 128)
         .transpose(0, 2, 1, 3)
         .reshape(ENC, N_PTS))
    out_t = _mlp(h, W0, b0.reshape(1, -1), W1, b1.reshape(1, -1),
                 W2, b2.reshape(-1, 1))
    return out_t.T
